# baseline (speedup 1.0000x reference)
import functools

import jax
import jax.numpy as jnp
from jax import lax
from jax.experimental import pallas as pl
from jax.experimental.pallas import tpu as pltpu
from jax.experimental.pallas import tpu_sc as plsc


_CB = 4096


def _pack_transpose_body(E, CB, x_ref, il_ref, ir_ref, out_ref):
    f32 = jnp.float32
    te = jax.lax.dot_general(x_ref[:, 0:CB // 2], il_ref[...],
                             (((0,), (0,)), ((), ())),
                             preferred_element_type=f32)
    to = jax.lax.dot_general(x_ref[:, CB // 2:CB], ir_ref[...],
                             (((0,), (0,)), ((), ())),
                             preferred_element_type=f32)
    out_ref[...] = te + to


def _run_pad_transpose(embT):
    E, V = embT.shape
    CB = _CB
    nb = (V + CB - 1) // CB
    zeroE = jnp.zeros((E, E), jnp.float32)
    il = jnp.concatenate([jnp.eye(E, dtype=jnp.float32), zeroE], axis=1)
    ir = jnp.concatenate([zeroE, jnp.eye(E, dtype=jnp.float32)], axis=1)
    return pl.pallas_call(
        functools.partial(_pack_transpose_body, E, CB),
        grid=(nb,),
        in_specs=[
            pl.BlockSpec((E, CB), lambda j: (0, j)),
            pl.BlockSpec((E, 2 * E), lambda j: (0, 0)),
            pl.BlockSpec((E, 2 * E), lambda j: (0, 0)),
        ],
        out_specs=pl.BlockSpec((CB // 2, 2 * E), lambda j: (j, 0)),
        out_shape=jax.ShapeDtypeStruct((nb * CB // 2, 2 * E), jnp.float32),
        compiler_params=pltpu.CompilerParams(
            dimension_semantics=("arbitrary",),
        ),
    )(embT, il, ir)


def _make_sc_gather(V2, D2, N):
    info = plsc.get_sparse_core_info()
    NC, NS = info.num_cores, info.num_subcores
    NW = NC * NS
    assert N % NW == 0
    n_per_w = N // NW
    CHUNK = 400
    assert n_per_w % CHUNK == 0
    n_chunks = n_per_w // CHUNK

    mesh = plsc.VectorSubcoreMesh(core_axis_name="c", subcore_axis_name="s")

    @functools.partial(
        pl.kernel,
        out_type=jax.ShapeDtypeStruct((N, D2), jnp.float32),
        mesh=mesh,
        scratch_types=[
            pltpu.VMEM((n_per_w,), jnp.int32),
            pltpu.VMEM((CHUNK, D2), jnp.float32),
            pltpu.VMEM((CHUNK, D2), jnp.float32),
            pltpu.SemaphoreType.DMA,
            pltpu.SemaphoreType.DMA,
        ],
    )
    def gather(table_hbm, idx_hbm, out_hbm, idx_v, rows_a, rows_b, sem_a, sem_b):
        wid = lax.axis_index("s") * NC + lax.axis_index("c")
        base = wid * n_per_w
        pltpu.sync_copy(idx_hbm.at[pl.ds(base, n_per_w)], idx_v)
        bufs = ((rows_a, sem_a), (rows_b, sem_b))
        copies = []
        for c in range(n_chunks):
            rows_v, sem = bufs[c % 2]
            if c >= 2:
                copies[c - 2].wait()
                pltpu.sync_copy(rows_v, out_hbm.at[pl.ds(base + (c - 2) * CHUNK, CHUNK)])
            copies.append(pltpu.async_copy(
                table_hbm.at[idx_v.at[pl.ds(c * CHUNK, CHUNK)]], rows_v, sem))
        for c in range(n_chunks - 2, n_chunks):
            rows_v, sem = bufs[c % 2]
            copies[c].wait()
            pltpu.sync_copy(rows_v, out_hbm.at[pl.ds(base + c * CHUNK, CHUNK)])

    return gather


def _mm(a, b):
    return jax.lax.dot_general(a, b, (((1,), (0,)), ((), ())),
                               preferred_element_type=jnp.float32)


def _lstm_body(H, TL, xf_ref, xb_ref, pf_ref, pb_ref, wih2_f, whh_f, b_f,
               wih2_b, whh_b, b_b, hsf_ref, hsb_ref, hf, cf, hb, cb):
    j = pl.program_id(0)

    @pl.when(j == 0)
    def _():
        hf[...] = jnp.zeros_like(hf)
        cf[...] = jnp.zeros_like(cf)
        hb[...] = jnp.zeros_like(hb)
        cb[...] = jnp.zeros_like(cb)

    def substep(xw, parb, w_ih2, w_hh, b, h, c):
        E2 = xw.shape[-1]
        lane = lax.broadcasted_iota(jnp.int32, (1, E2), 1)
        mask = jnp.where(lane < E2 // 2, 1.0 - parb, parb)
        gates = jax.lax.dot_general(
            w_ih2[...], xw * mask, (((1,), (1,)), ((), ())),
            preferred_element_type=jnp.float32) \
            + _mm(w_hh[...], h) + b[...]
        sig = lambda z: 0.5 * jnp.tanh(0.5 * z) + 0.5
        i = sig(gates[0 * H:1 * H])
        f = sig(gates[1 * H:2 * H])
        g = jnp.tanh(gates[2 * H:3 * H])
        o = sig(gates[3 * H:4 * H])
        c_new = f * c + i * g
        h_new = o * jnp.tanh(c_new)
        return h_new, c_new

    h_f, c_f = hf[...], cf[...]
    h_b, c_b = hb[...], cb[...]
    for k in range(TL):
        h_f, c_f = substep(xf_ref[k], pf_ref[k], wih2_f, whh_f, b_f, h_f, c_f)
        hsf_ref[k] = h_f
        h_b, c_b = substep(xb_ref[TL - 1 - k], pb_ref[TL - 1 - k],
                           wih2_b, whh_b, b_b, h_b, c_b)
        hsb_ref[TL - 1 - k] = h_b
    hf[...], cf[...] = h_f, c_f
    hb[...], cb[...] = h_b, c_b


def _run_lstm(xw, par, wih2_f, whh_f, b_f, wih2_b, whh_b, b_b, interpret=False):
    L, B, E2 = xw.shape
    H = whh_f.shape[1]
    TL = 8
    NB = L // TL
    const = lambda shape: pl.BlockSpec(shape, lambda j: (0,) * len(shape))
    return pl.pallas_call(
        functools.partial(_lstm_body, H, TL),
        grid=(NB,),
        in_specs=[
            pl.BlockSpec((TL, B, E2), lambda j: (j, 0, 0)),
            pl.BlockSpec((TL, B, E2), lambda j: (NB - 1 - j, 0, 0)),
            pl.BlockSpec((TL, B, 1), lambda j: (j, 0, 0)),
            pl.BlockSpec((TL, B, 1), lambda j: (NB - 1 - j, 0, 0)),
            const((4 * H, E2)), const((4 * H, H)), const((4 * H, 1)),
            const((4 * H, E2)), const((4 * H, H)), const((4 * H, 1)),
        ],
        out_specs=[
            pl.BlockSpec((TL, H, B), lambda j: (j, 0, 0)),
            pl.BlockSpec((TL, H, B), lambda j: (NB - 1 - j, 0, 0)),
        ],
        out_shape=[
            jax.ShapeDtypeStruct((L, H, B), jnp.float32),
            jax.ShapeDtypeStruct((L, H, B), jnp.float32),
        ],
        scratch_shapes=[pltpu.VMEM((H, B), jnp.float32)] * 4,
        compiler_params=pltpu.CompilerParams(
            dimension_semantics=("arbitrary",),
        ),
        interpret=interpret,
    )(xw, xw, par, par, wih2_f, whh_f, b_f, wih2_b, whh_b, b_b)


def _attn_body(NB, TA, hsf_ref, hsb_ref, hnf_ref, hnb_ref, wout, bo,
               out_ref, m_s, d_s, accf_s, accb_s):
    j = pl.program_id(0)

    @pl.when(j == 0)
    def _():
        m_s[...] = jnp.full_like(m_s, -jnp.inf)
        d_s[...] = jnp.zeros_like(d_s)
        accf_s[...] = jnp.zeros_like(accf_s)
        accb_s[...] = jnp.zeros_like(accb_s)

    hnf = hnf_ref[0]
    hnb = hnb_ref[0]
    m, d = m_s[...], d_s[...]
    accf, accb = accf_s[...], accb_s[...]
    for k in range(TA):
        hf = hsf_ref[k]
        hb = hsb_ref[k]
        s = (jnp.sum(hf * hnf, axis=0, keepdims=True)
             + jnp.sum(hb * hnb, axis=0, keepdims=True))
        m_new = jnp.maximum(m, s)
        alpha = jnp.exp(m - m_new)
        p = jnp.exp(s - m_new)
        d = d * alpha + p
        accf = accf * alpha + p * hf
        accb = accb * alpha + p * hb
        m = m_new
    m_s[...], d_s[...] = m, d
    accf_s[...], accb_s[...] = accf, accb

    @pl.when(j == NB - 1)
    def _():
        inv = 1.0 / d
        ctx = jnp.concatenate([accf * inv, accb * inv], axis=0)
        out_ref[...] = jax.lax.dot_general(
            ctx, wout[...], (((0,), (1,)), ((), ())),
            preferred_element_type=jnp.float32) + bo[...]


def _run_attn(hsf, hsb, wout, bo, interpret=False):
    L, H, B = hsf.shape
    C = wout.shape[0]
    TA = 8
    NB = L // TA
    const = lambda shape: pl.BlockSpec(shape, lambda j: (0,) * len(shape))
    return pl.pallas_call(
        functools.partial(_attn_body, NB, TA),
        grid=(NB,),
        in_specs=[
            pl.BlockSpec((TA, H, B), lambda j: (j, 0, 0)),
            pl.BlockSpec((TA, H, B), lambda j: (j, 0, 0)),
            pl.BlockSpec((1, H, B), lambda j: (0, 0, 0)),
            pl.BlockSpec((1, H, B), lambda j: (L - 1, 0, 0)),
            const((C, 2 * H)), const((1, C)),
        ],
        out_specs=pl.BlockSpec((B, C), lambda j: (0, 0)),
        out_shape=jax.ShapeDtypeStruct((B, C), jnp.float32),
        scratch_shapes=[
            pltpu.VMEM((1, B), jnp.float32),
            pltpu.VMEM((1, B), jnp.float32),
            pltpu.VMEM((H, B), jnp.float32),
            pltpu.VMEM((H, B), jnp.float32),
        ],
        compiler_params=pltpu.CompilerParams(
            dimension_semantics=("arbitrary",),
        ),
        interpret=interpret,
    )(hsf, hsb, hsb, hsf, wout, bo)


def kernel(batch, emb, Wih_f, Whh_f, bih_f, bhh_f, Wih_b, Whh_b, bih_b, bhh_b,
           W_out, b_out):
    B, L = batch.shape
    V, E = emb.shape
    H = Whh_f.shape[1]
    C = W_out.shape[0]

    idx = batch.astype(jnp.int32).T.reshape(-1)
    emb2 = _run_pad_transpose(emb.T)
    half = _CB // 2
    idx_row = ((idx // _CB) * half) + (idx % half)
    x_wide = _make_sc_gather(emb2.shape[0], 2 * E, L * B)(emb2, idx_row)
    xw = x_wide.reshape(L, B, 2 * E)
    par = ((idx % _CB) // half).astype(jnp.float32).reshape(L, B, 1)

    b_f = (bih_f + bhh_f).reshape(4 * H, 1)
    b_b = (bih_b + bhh_b).reshape(4 * H, 1)
    wih2_f = jnp.concatenate([Wih_f, Wih_f], axis=1)
    wih2_b = jnp.concatenate([Wih_b, Wih_b], axis=1)
    hsf, hsb = _run_lstm(xw, par, wih2_f, Whh_f, b_f, wih2_b, Whh_b, b_b)
    return _run_attn(hsf, hsb, W_out, b_out.reshape(1, C))
